# baseline (speedup 1.0000x reference)
import functools

import jax
import jax.numpy as jnp
from jax.experimental import pallas as pl
from jax.experimental.pallas import tpu as pltpu

B, L, D = 16, 4096, 1024
BL = 512


def _tc_body(xlen_ref, x_ref, o_ref):
    b = pl.program_id(0)
    j = pl.program_id(1)
    rows = jax.lax.broadcasted_iota(jnp.int32, (1, BL, D), 1) + j * BL
    o_ref[...] = jnp.where(rows < xlen_ref[b], x_ref[...], 0.0)


def kernel(x, x_len):
    xl = x_len.astype(jnp.int32)
    out = pl.pallas_call(
        _tc_body,
        grid_spec=pltpu.PrefetchScalarGridSpec(
            num_scalar_prefetch=1,
            grid=(B, L // BL),
            in_specs=[pl.BlockSpec((1, BL, D), lambda b, j, xl: (b, j, 0))],
            out_specs=pl.BlockSpec((1, BL, D), lambda b, j, xl: (b, j, 0)),
        ),
        out_shape=jax.ShapeDtypeStruct((B, L, D), jnp.float32),
    )(xl, x)
    return out
